# Initial kernel scaffold; baseline (speedup 1.0000x reference)
import jax
import jax.numpy as jnp
from jax.experimental import pallas as pl


def kernel(inputs, class_embed, pos_table):
    raise NotImplementedError("write your pallas kernel here")


# speedup vs baseline: 1.0659x; 1.0659x over previous
import jax
import jax.numpy as jnp
from jax.experimental import pallas as pl

D_MODEL = 768
N_PATCHES = 576
N_TOT = N_PATCHES + 1
BATCH = 128
BB = 4


def _body(in_ref, cls_ref, pos_ref, out_ref):
    cls_rows = jnp.broadcast_to(cls_ref[...], (BB, 1, D_MODEL))
    x = jnp.concatenate([cls_rows, in_ref[...]], axis=1)
    out_ref[...] = x + pos_ref[...][None]


def kernel(inputs, class_embed, pos_table):
    grid = (BATCH // BB,)
    return pl.pallas_call(
        _body,
        grid=grid,
        in_specs=[
            pl.BlockSpec((BB, N_PATCHES, D_MODEL), lambda i: (i, 0, 0)),
            pl.BlockSpec((1, 1, D_MODEL), lambda i: (0, 0, 0)),
            pl.BlockSpec((N_TOT, D_MODEL), lambda i: (0, 0)),
        ],
        out_specs=pl.BlockSpec((BB, N_TOT, D_MODEL), lambda i: (i, 0, 0)),
        out_shape=jax.ShapeDtypeStruct((BATCH, N_TOT, D_MODEL), jnp.float32),
    )(inputs, class_embed, pos_table)


# speedup vs baseline: 1.0741x; 1.0077x over previous
import jax
import jax.numpy as jnp
from jax.experimental import pallas as pl

D_MODEL = 768
N_PATCHES = 576
N_TOT = N_PATCHES + 1
BATCH = 128
BB = 8


def _body(in_ref, cls_ref, pos_ref, out_ref):
    cls_rows = jnp.broadcast_to(cls_ref[...], (BB, 1, D_MODEL))
    x = jnp.concatenate([cls_rows, in_ref[...]], axis=1)
    out_ref[...] = x + pos_ref[...][None]


def kernel(inputs, class_embed, pos_table):
    grid = (BATCH // BB,)
    return pl.pallas_call(
        _body,
        grid=grid,
        in_specs=[
            pl.BlockSpec((BB, N_PATCHES, D_MODEL), lambda i: (i, 0, 0)),
            pl.BlockSpec((1, 1, D_MODEL), lambda i: (0, 0, 0)),
            pl.BlockSpec((N_TOT, D_MODEL), lambda i: (0, 0)),
        ],
        out_specs=pl.BlockSpec((BB, N_TOT, D_MODEL), lambda i: (i, 0, 0)),
        out_shape=jax.ShapeDtypeStruct((BATCH, N_TOT, D_MODEL), jnp.float32),
    )(inputs, class_embed, pos_table)
